# baseline (speedup 1.0000x reference)
import jax
import jax.numpy as jnp
from jax.experimental import pallas as pl
from jax.experimental.pallas import tpu as pltpu

_NNZ = 1
_N_UNIQUE = 1
_DENSE = 6
_EXPECTED_SIZE = (1, 2, 3)


def _coalesce_kernel(idx_ref, vals_ref, out_ref, res_ref):
    rows = jax.lax.broadcasted_iota(jnp.int32, (_N_UNIQUE, _DENSE), 0)
    acc = jnp.zeros((_N_UNIQUE, _DENSE), jnp.float32)
    for i in range(_NNZ):
        seg = idx_ref[0, i]
        acc = acc + jnp.where(rows == seg, vals_ref[i, :][None, :], 0.0)
    out_ref[...] = acc
    coalesced_size = (_N_UNIQUE,) + _EXPECTED_SIZE[1:]
    res_ref[0, 0] = jnp.int32(1 if coalesced_size == _EXPECTED_SIZE else 0)


def kernel(x):
    del x
    indices = jnp.zeros((1, _NNZ), dtype=jnp.int32)
    values = jnp.ones((_NNZ, _DENSE), dtype=jnp.float32)
    _, res = pl.pallas_call(
        _coalesce_kernel,
        in_specs=[
            pl.BlockSpec(memory_space=pltpu.SMEM),
            pl.BlockSpec(memory_space=pltpu.VMEM),
        ],
        out_specs=[
            pl.BlockSpec(memory_space=pltpu.VMEM),
            pl.BlockSpec(memory_space=pltpu.SMEM),
        ],
        out_shape=[
            jax.ShapeDtypeStruct((_N_UNIQUE, _DENSE), jnp.float32),
            jax.ShapeDtypeStruct((1, 1), jnp.int32),
        ],
    )(indices, values)
    return res.reshape(1).astype(jnp.int64)


# speedup vs baseline: 1.0125x; 1.0125x over previous
import jax
import jax.numpy as jnp
from jax.experimental import pallas as pl
from jax.experimental.pallas import tpu as pltpu

_NNZ = 1
_N_UNIQUE = 1
_DENSE = 6
_EXPECTED_SIZE = (1, 2, 3)


def _coalesce_kernel(idx_ref, vals_ref, out_ref, res_ref):
    for s in range(_N_UNIQUE):
        for j in range(_DENSE):
            acc = jnp.float32(0.0)
            for i in range(_NNZ):
                acc = acc + jnp.where(idx_ref[0, i] == s, vals_ref[i, j], 0.0)
            out_ref[s, j] = acc
    coalesced_size = (_N_UNIQUE,) + _EXPECTED_SIZE[1:]
    res_ref[0, 0] = jnp.int32(1 if coalesced_size == _EXPECTED_SIZE else 0)


def kernel(x):
    del x
    indices = jnp.zeros((1, _NNZ), dtype=jnp.int32)
    values = jnp.ones((_NNZ, _DENSE), dtype=jnp.float32)
    _, res = pl.pallas_call(
        _coalesce_kernel,
        in_specs=[
            pl.BlockSpec(memory_space=pltpu.SMEM),
            pl.BlockSpec(memory_space=pltpu.SMEM),
        ],
        out_specs=[
            pl.BlockSpec(memory_space=pltpu.SMEM),
            pl.BlockSpec(memory_space=pltpu.SMEM),
        ],
        out_shape=[
            jax.ShapeDtypeStruct((_N_UNIQUE, _DENSE), jnp.float32),
            jax.ShapeDtypeStruct((1, 1), jnp.int32),
        ],
    )(indices, values)
    return res.reshape(1).astype(jnp.int64)
